# baseline (speedup 1.0000x reference)
import jax
import jax.numpy as jnp
from jax import lax
from jax.experimental import pallas as pl
from jax.experimental.pallas import tpu as pltpu
from jax.experimental.pallas import tpu_sc as plsc

_RADIUS = 0.2
_NSAMPLE = 32
_B, _N, _NP, _C = 8, 4096, 1024, 64
_NCH = 3 + _C
_NW = 32
_QPW = (_B * _NP) // _NW
_TPB = _NP // _QPW
_GSZ = _QPW * _NSAMPLE
_CHSTRIDE = _NP * _NSAMPLE
_SELSTRIDE = _NSAMPLE
_UNROLL = 16
_CGRP = 4


def _sc_body(xyz_hbm, q_hbm, pts_hbm, out_hbm,
             xt, yt, zt, qv, idxs, sel, tblA, tblB, stg0, stg1, stg2, stg3,
             sem_in0, sem_in1, sem_out):
    wid = lax.axis_index("s") * 2 + lax.axis_index("c")
    b = wid // _TPB
    p0 = (wid % _TPB) * _QPW

    pltpu.sync_copy(xyz_hbm.at[b * 3 + 0], xt)
    pltpu.sync_copy(xyz_hbm.at[b * 3 + 1], yt)
    pltpu.sync_copy(xyz_hbm.at[b * 3 + 2], zt)
    for d in range(3):
        pltpu.sync_copy(q_hbm.at[b * 3 + d, pl.ds(p0, _QPW)],
                        qv.at[pl.ds(d * _QPW, _QPW)])

    lane = jnp.arange(16, dtype=jnp.int32)
    zeros16 = jnp.zeros((16,), jnp.int32)
    r2 = jnp.float32(_RADIUS * _RADIUS)

    _HQ = _QPW // 2
    _SELB = _HQ * _SELSTRIDE + _N + 128

    def per_pair(i, carry):
        pA, pB = i, i + _HQ
        qA = [jnp.full((16,), qv[pl.ds(d * _QPW + pA, 16)][0], jnp.float32)
              for d in range(3)]
        qB = [jnp.full((16,), qv[pl.ds(d * _QPW + pB, 16)][0], jnp.float32)
              for d in range(3)]
        sbA = i * _SELSTRIDE
        sbB = _SELB + i * _SELSTRIDE
        sel[pl.ds(sbA, 16)] = zeros16
        sel[pl.ds(sbB, 16)] = zeros16

        def cond(jc):
            j, cntA, cntB = jc
            return (j < _N // 16) & ((cntA < _NSAMPLE) | (cntB < _NSAMPLE))

        def wstep(jc):
            j, cntA, cntB = jc
            masksA, masksB = [], []
            for u in range(_UNROLL):
                off = (j + u) * 16
                xv = xt[pl.ds(off, 16)]
                yv = yt[pl.ds(off, 16)]
                zv = zt[pl.ds(off, 16)]
                dxa, dya, dza = xv - qA[0], yv - qA[1], zv - qA[2]
                masksA.append(dxa * dxa + dya * dya + dza * dza < r2)
                dxb, dyb, dzb = xv - qB[0], yv - qB[1], zv - qB[2]
                masksB.append(dxb * dxb + dyb * dyb + dzb * dzb < r2)
            pcsA = [plsc.all_reduce_population_count(m)[0] for m in masksA]
            pcsB = [plsc.all_reduce_population_count(m)[0] for m in masksB]
            offsA, offsB = [], []
            for u in range(_UNROLL):
                offsA.append(cntA)
                cntA = cntA + pcsA[u]
                offsB.append(cntB)
                cntB = cntB + pcsB[u]
            for u in range(_UNROLL):
                inds = (j + u) * 16 + lane
                plsc.store_compressed(sel.at[pl.ds(sbA + offsA[u], 16)],
                                      inds, mask=masksA[u])
                plsc.store_compressed(sel.at[pl.ds(sbB + offsB[u], 16)],
                                      inds, mask=masksB[u])
            return j + _UNROLL, cntA, cntB

        _, cntA, cntB = lax.while_loop(
            cond, wstep, (jnp.int32(0), jnp.int32(0), jnp.int32(0)))

        for p, sb, cnt, q in ((pA, sbA, cntA, qA), (pB, sbB, cntB, qB)):
            base = p * _NSAMPLE
            first = jnp.full((16,), sel[pl.ds(sb, 16)][0], jnp.int32)
            for k in range(2):
                have = (lane + 16 * k) < cnt
                iv = jnp.where(have, sel[pl.ds(sb + 16 * k, 16)], first)
                idxs[pl.ds(base + 16 * k, 16)] = iv
                stg0[pl.ds(base + 16 * k, 16)] = (
                    plsc.load_gather(xt, [iv]) - q[0])
                stg1[pl.ds(base + 16 * k, 16)] = (
                    plsc.load_gather(yt, [iv]) - q[1])
                stg2[pl.ds(base + 16 * k, 16)] = (
                    plsc.load_gather(zt, [iv]) - q[2])
        return carry

    lax.fori_loop(0, _HQ, per_pair, jnp.int32(0))


    out_base = (b * _NCH) * _CHSTRIDE + p0 * _NSAMPLE
    stg = [stg0, stg1, stg2, stg3]
    for d in range(3):
        pltpu.sync_copy(stg[d],
                        out_hbm.at[pl.ds(out_base + d * _CHSTRIDE, _GSZ)])

    tbl = [tblA, tblB]
    sem_in = [sem_in0, sem_in1]
    rsplat = [jnp.full((16,), r, jnp.int32) for r in range(_CGRP)]

    def in_grp(g):
        return pts_hbm.at[pl.ds(b * _C + _CGRP * g, _CGRP)]

    def out_slab(g, r):
        return out_hbm.at[pl.ds(out_base + (3 + _CGRP * g + r) * _CHSTRIDE,
                                _GSZ)]

    def gather_group(tref):
        def gg(i, cc):
            iv = idxs[pl.ds(i * 16, 16)]
            for r in range(_CGRP):
                stg[r][pl.ds(i * 16, 16)] = plsc.load_gather(tref,
                                                             [rsplat[r], iv])
            return cc
        lax.fori_loop(0, _GSZ // 16, gg, jnp.int32(0), unroll=4)

    _NG = _C // _CGRP
    pltpu.async_copy(in_grp(0), tblA, sem_in[0])
    pltpu.async_copy(in_grp(1), tblB, sem_in[1])

    pltpu.make_async_copy(in_grp(0), tblA, sem_in[0]).wait()
    gather_group(tblA)
    pltpu.async_copy(in_grp(2), tblA, sem_in[0])
    for r in range(_CGRP):
        pltpu.async_copy(stg[r], out_slab(0, r), sem_out)

    def grp_pair(cc, carry):
        for par in range(2):
            g = 2 * cc + par
            tref = tbl[par]
            pltpu.make_async_copy(in_grp(g), tref, sem_in[par]).wait()
            for r in range(_CGRP):
                pltpu.make_async_copy(stg[r], out_slab(g - 1, r),
                                      sem_out).wait()
            gather_group(tref)
            gn = jnp.minimum(g + 2, _NG - 2 + par)
            pltpu.async_copy(in_grp(gn), tref, sem_in[par])
            for r in range(_CGRP):
                pltpu.async_copy(stg[r], out_slab(g, r), sem_out)
        return carry

    pltpu.make_async_copy(in_grp(1), tblB, sem_in[1]).wait()
    for r in range(_CGRP):
        pltpu.make_async_copy(stg[r], out_slab(0, r), sem_out).wait()
    gather_group(tblB)
    pltpu.async_copy(in_grp(3), tblB, sem_in[1])
    for r in range(_CGRP):
        pltpu.async_copy(stg[r], out_slab(1, r), sem_out)

    lax.fori_loop(1, _NG // 2, grp_pair, jnp.int32(0))

    for r in range(_CGRP):
        pltpu.make_async_copy(stg[r], out_slab(_NG - 1, r), sem_out).wait()
    pltpu.make_async_copy(in_grp(_NG - 2), tblA, sem_in[0]).wait()
    pltpu.make_async_copy(in_grp(_NG - 1), tblB, sem_in[1]).wait()


@jax.jit
def kernel(xyz, new_xyz, points):
    xt = jnp.transpose(xyz, (0, 2, 1)).reshape(_B * 3, _N)
    qt = jnp.transpose(new_xyz, (0, 2, 1)).reshape(_B * 3, _NP)
    pts = points.reshape(_B * _C, _N)
    fn = pl.kernel(
        _sc_body,
        out_type=jax.ShapeDtypeStruct((_B * _NCH * _NP * _NSAMPLE,),
                                      jnp.float32),
        mesh=plsc.VectorSubcoreMesh(core_axis_name="c", subcore_axis_name="s"),
        compiler_params=pltpu.CompilerParams(needs_layout_passes=False),
        scratch_types=[
            pltpu.VMEM((_N,), jnp.float32),
            pltpu.VMEM((_N,), jnp.float32),
            pltpu.VMEM((_N,), jnp.float32),
            pltpu.VMEM((3 * _QPW + 16,), jnp.float32),
            pltpu.VMEM((_GSZ,), jnp.int32),
            pltpu.VMEM((2 * ((_QPW // 2) * _SELSTRIDE + _N + 128),), jnp.int32),
            pltpu.VMEM((_CGRP, _N), jnp.float32),
            pltpu.VMEM((_CGRP, _N), jnp.float32),
            pltpu.VMEM((_GSZ,), jnp.float32),
            pltpu.VMEM((_GSZ,), jnp.float32),
            pltpu.VMEM((_GSZ,), jnp.float32),
            pltpu.VMEM((_GSZ,), jnp.float32),
            pltpu.SemaphoreType.DMA,
            pltpu.SemaphoreType.DMA,
            pltpu.SemaphoreType.DMA,
        ],
    )
    out = fn(xt, qt, pts)
    return out.reshape(_B, _NCH, _NP, _NSAMPLE)


# speedup vs baseline: 1.0250x; 1.0250x over previous
import jax
import jax.numpy as jnp
from jax import lax
from jax.experimental import pallas as pl
from jax.experimental.pallas import tpu as pltpu
from jax.experimental.pallas import tpu_sc as plsc

_RADIUS = 0.2
_NSAMPLE = 32
_B, _N, _NP, _C = 8, 4096, 1024, 64
_NCH = 3 + _C
_NW = 32
_QPW = (_B * _NP) // _NW
_TPB = _NP // _QPW
_GSZ = _QPW * _NSAMPLE
_CHSTRIDE = _NP * _NSAMPLE
_SELSTRIDE = _NSAMPLE
_UNROLL = 16
_CGRP = 4


def _sc_body(xyz_hbm, q_hbm, pts_hbm, out_hbm,
             xt, yt, zt, qv, idxs, sel, tblA, tblB, stg0, stg1, stg2, stg3,
             sem_in0, sem_in1, sem_out):
    wid = lax.axis_index("s") * 2 + lax.axis_index("c")
    b = wid // _TPB
    p0 = (wid % _TPB) * _QPW

    pltpu.sync_copy(xyz_hbm.at[b * 3 + 0], xt)
    pltpu.sync_copy(xyz_hbm.at[b * 3 + 1], yt)
    pltpu.sync_copy(xyz_hbm.at[b * 3 + 2], zt)
    for d in range(3):
        pltpu.sync_copy(q_hbm.at[b * 3 + d, pl.ds(p0, _QPW)],
                        qv.at[pl.ds(d * _QPW, _QPW)])

    lane = jnp.arange(16, dtype=jnp.int32)
    zeros16 = jnp.zeros((16,), jnp.int32)
    r2 = jnp.float32(_RADIUS * _RADIUS)

    def per_query(p, carry):
        qx = jnp.full((16,), qv[pl.ds(p, 16)][0], jnp.float32)
        qy = jnp.full((16,), qv[pl.ds(_QPW + p, 16)][0], jnp.float32)
        qz = jnp.full((16,), qv[pl.ds(2 * _QPW + p, 16)][0], jnp.float32)
        base = p * _NSAMPLE
        sbase = p * _SELSTRIDE
        sel[pl.ds(sbase, 16)] = zeros16

        def cond(jc):
            j, cnt = jc
            return (j < _N // 16) & (cnt < _NSAMPLE)

        def wstep(jc):
            j, cnt = jc
            masks = []
            for u in range(_UNROLL):
                off = (j + u) * 16
                dx = xt[pl.ds(off, 16)] - qx
                dy = yt[pl.ds(off, 16)] - qy
                dz = zt[pl.ds(off, 16)] - qz
                d2 = dx * dx + dy * dy + dz * dz
                masks.append(d2 < r2)
            pcs = [plsc.all_reduce_population_count(m)[0] for m in masks]
            offs = []
            for u in range(_UNROLL):
                offs.append(cnt)
                cnt = cnt + pcs[u]
            for u in range(_UNROLL):
                plsc.store_compressed(sel.at[pl.ds(sbase + offs[u], 16)],
                                      (j + u) * 16 + lane, mask=masks[u])
            return j + _UNROLL, cnt

        _, cnt = lax.while_loop(cond, wstep, (jnp.int32(0), jnp.int32(0)))

        first = jnp.full((16,), sel[pl.ds(sbase, 16)][0], jnp.int32)
        for k in range(2):
            have = (lane + 16 * k) < cnt
            iv = jnp.where(have, sel[pl.ds(sbase + 16 * k, 16)], first)
            idxs[pl.ds(base + 16 * k, 16)] = iv
            stg0[pl.ds(base + 16 * k, 16)] = plsc.load_gather(xt, [iv]) - qx
            stg1[pl.ds(base + 16 * k, 16)] = plsc.load_gather(yt, [iv]) - qy
            stg2[pl.ds(base + 16 * k, 16)] = plsc.load_gather(zt, [iv]) - qz
        return carry

    lax.fori_loop(0, _QPW, per_query, jnp.int32(0))

    out_base = (b * _NCH) * _CHSTRIDE + p0 * _NSAMPLE
    stg = [stg0, stg1, stg2, stg3]
    for d in range(3):
        pltpu.sync_copy(stg[d],
                        out_hbm.at[pl.ds(out_base + d * _CHSTRIDE, _GSZ)])

    tbl = [tblA, tblB]
    sem_in = [sem_in0, sem_in1]
    rsplat = [jnp.full((16,), r, jnp.int32) for r in range(_CGRP)]

    def in_grp(g):
        return pts_hbm.at[pl.ds(b * _C + _CGRP * g, _CGRP)]

    def out_slab(g, r):
        return out_hbm.at[pl.ds(out_base + (3 + _CGRP * g + r) * _CHSTRIDE,
                                _GSZ)]

    def gather_group(tref):
        def gg(i, cc):
            iv = idxs[pl.ds(i * 16, 16)]
            for r in range(_CGRP):
                stg[r][pl.ds(i * 16, 16)] = plsc.load_gather(tref,
                                                             [rsplat[r], iv])
            return cc
        lax.fori_loop(0, _GSZ // 16, gg, jnp.int32(0), unroll=8)

    _NG = _C // _CGRP
    pltpu.async_copy(in_grp(0), tblA, sem_in[0])
    pltpu.async_copy(in_grp(1), tblB, sem_in[1])

    pltpu.make_async_copy(in_grp(0), tblA, sem_in[0]).wait()
    gather_group(tblA)
    pltpu.async_copy(in_grp(2), tblA, sem_in[0])
    for r in range(_CGRP):
        pltpu.async_copy(stg[r], out_slab(0, r), sem_out)

    def grp_pair(cc, carry):
        for par in range(2):
            g = 2 * cc + par
            tref = tbl[par]
            pltpu.make_async_copy(in_grp(g), tref, sem_in[par]).wait()
            for r in range(_CGRP):
                pltpu.make_async_copy(stg[r], out_slab(g - 1, r),
                                      sem_out).wait()
            gather_group(tref)
            gn = jnp.minimum(g + 2, _NG - 2 + par)
            pltpu.async_copy(in_grp(gn), tref, sem_in[par])
            for r in range(_CGRP):
                pltpu.async_copy(stg[r], out_slab(g, r), sem_out)
        return carry

    pltpu.make_async_copy(in_grp(1), tblB, sem_in[1]).wait()
    for r in range(_CGRP):
        pltpu.make_async_copy(stg[r], out_slab(0, r), sem_out).wait()
    gather_group(tblB)
    pltpu.async_copy(in_grp(3), tblB, sem_in[1])
    for r in range(_CGRP):
        pltpu.async_copy(stg[r], out_slab(1, r), sem_out)

    lax.fori_loop(1, _NG // 2, grp_pair, jnp.int32(0))

    for r in range(_CGRP):
        pltpu.make_async_copy(stg[r], out_slab(_NG - 1, r), sem_out).wait()
    pltpu.make_async_copy(in_grp(_NG - 2), tblA, sem_in[0]).wait()
    pltpu.make_async_copy(in_grp(_NG - 1), tblB, sem_in[1]).wait()


@jax.jit
def kernel(xyz, new_xyz, points):
    xt = jnp.transpose(xyz, (0, 2, 1)).reshape(_B * 3, _N)
    qt = jnp.transpose(new_xyz, (0, 2, 1)).reshape(_B * 3, _NP)
    pts = points.reshape(_B * _C, _N)
    fn = pl.kernel(
        _sc_body,
        out_type=jax.ShapeDtypeStruct((_B * _NCH * _NP * _NSAMPLE,),
                                      jnp.float32),
        mesh=plsc.VectorSubcoreMesh(core_axis_name="c", subcore_axis_name="s"),
        compiler_params=pltpu.CompilerParams(needs_layout_passes=False),
        scratch_types=[
            pltpu.VMEM((_N,), jnp.float32),
            pltpu.VMEM((_N,), jnp.float32),
            pltpu.VMEM((_N,), jnp.float32),
            pltpu.VMEM((3 * _QPW + 16,), jnp.float32),
            pltpu.VMEM((_GSZ,), jnp.int32),
            pltpu.VMEM((_GSZ + 16 * _UNROLL + 64,), jnp.int32),
            pltpu.VMEM((_CGRP, _N), jnp.float32),
            pltpu.VMEM((_CGRP, _N), jnp.float32),
            pltpu.VMEM((_GSZ,), jnp.float32),
            pltpu.VMEM((_GSZ,), jnp.float32),
            pltpu.VMEM((_GSZ,), jnp.float32),
            pltpu.VMEM((_GSZ,), jnp.float32),
            pltpu.SemaphoreType.DMA,
            pltpu.SemaphoreType.DMA,
            pltpu.SemaphoreType.DMA,
        ],
    )
    out = fn(xt, qt, pts)
    return out.reshape(_B, _NCH, _NP, _NSAMPLE)
